# baseline (speedup 1.0000x reference)
import functools

import jax
import jax.numpy as jnp
from jax import lax
from jax.experimental import pallas as pl
from jax.experimental.pallas import tpu as pltpu
from jax.experimental.pallas import tpu_sc as plsc

N = 10000
NPAD = 10240
E = 160000
T = 16
CK = 100
NC = (E // T) // CK
NB = 10
RPT = NPAD // T
D2 = 128
CKP = 128
NCP = 10240 // CKP

_mesh = plsc.VectorSubcoreMesh(core_axis_name="c", subcore_axis_name="s")


@functools.partial(
    pl.kernel,
    mesh=_mesh,
    out_type=[jax.ShapeDtypeStruct((NPAD,), jnp.float32)] * 2,
    scratch_types=[
        pltpu.VMEM((NC, CK), jnp.int32),
        pltpu.VMEM((CK,), jnp.float32),
        pltpu.VMEM((RPT,), jnp.float32),
        pltpu.VMEM_SHARED((NPAD,), jnp.float32),
    ],
)
def _deg_kernel(src_r, dst_r, dego, degi, idxv, onesv, zv, acc):
    c = lax.axis_index("c")
    s = lax.axis_index("s")

    def fill(ref, n, val):
        def body(i, carry):
            ref[pl.ds(i * 16, 16)] = jnp.full((16,), val, jnp.float32)
            return carry
        lax.fori_loop(0, n // 16, body, 0)
        if n % 16:
            ref[pl.ds(n - 16, 16)] = jnp.full((16,), val, jnp.float32)

    fill(onesv, CK, 1.0)
    fill(zv, RPT, 0.0)
    pltpu.sync_copy(zv, acc.at[pl.ds(s * RPT, RPT)])
    plsc.subcore_barrier()

    def run(e_r, out_ref):
        def body(g, carry):
            pltpu.sync_copy(onesv, acc.at[idxv.at[g]], add=True)
            return carry

        pltpu.sync_copy(e_r.at[s], idxv)
        lax.fori_loop(0, NC, body, 0)
        plsc.subcore_barrier()
        pltpu.sync_copy(acc.at[pl.ds(s * RPT, RPT)],
                        out_ref.at[pl.ds(s * RPT, RPT)])

    @pl.when(c == 0)
    def _():
        run(src_r, dego)

    @pl.when(c == 1)
    def _():
        run(dst_r, degi)


def _make_prop(split, d2=D2):
    nblk = (NCP if split == "feat" else NCP // 2) // NB

    _deco = functools.partial(
        pl.kernel,
        mesh=_mesh,
        compiler_params=pltpu.CompilerParams(use_tc_tiling_on_sc=False),
        out_type=[jax.ShapeDtypeStruct((NPAD, d2), jnp.float32)] * 2,
        scratch_types=[
            pltpu.VMEM((NB, CKP), jnp.int32),
            pltpu.VMEM((NB, CKP), jnp.int32),
            pltpu.VMEM((CKP, d2), jnp.float32),
            pltpu.VMEM((CKP, d2), jnp.float32),
            pltpu.VMEM_SHARED((NPAD, d2), jnp.float32),
            pltpu.SemaphoreType.DMA,
            pltpu.SemaphoreType.DMA,
        ],
    )

    def body_common(split_ms, src_r, dst_r, o0, o1, srcv, dstv, buf0, buf1,
                    acc, sem0, sem1):
        c = lax.axis_index("c")
        s = lax.axis_index("s")

        def zrow(i, carry):
            def zcol(j, carry2):
                buf0[i, pl.ds(j * 16, 16)] = jnp.zeros((16,), jnp.float32)
                return carry2
            lax.fori_loop(0, d2 // 16, zcol, 0)
            return carry

        lax.fori_loop(0, CKP, zrow, 0)
        base = s * RPT
        for q in range(RPT // CKP):
            pltpu.sync_copy(buf0, acc.at[pl.ds(base + q * CKP, CKP)])
        plsc.subcore_barrier()

        def run(m_ref, out_ref, blk0):
            def gather(j, buf, sem):
                return pltpu.async_copy(m_ref.at[srcv.at[j]], buf, sem)

            def scat(j, buf, sem):
                return pltpu.async_copy(buf, acc.at[dstv.at[j]], sem,
                                        add=True)

            for blk in range(nblk):
                pltpu.sync_copy(src_r.at[s, pl.ds((blk0 + blk) * NB, NB)], srcv)
                pltpu.sync_copy(dst_r.at[s, pl.ds((blk0 + blk) * NB, NB)], dstv)
                gather(0, buf0, sem0)
                gather(1, buf1, sem1)

                def body(i, carry):
                    j0 = 2 * i
                    j1 = j0 + 1
                    pltpu.make_async_copy(m_ref.at[srcv.at[j0]], buf0,
                                          sem0).wait()
                    scat(j0, buf0, sem0)
                    pltpu.make_async_copy(m_ref.at[srcv.at[j1]], buf1,
                                          sem1).wait()
                    scat(j1, buf1, sem1)
                    pltpu.make_async_copy(buf0, acc.at[dstv.at[j0]],
                                          sem0).wait()

                    @pl.when(j0 + 2 < NB)
                    def _():
                        gather(j0 + 2, buf0, sem0)

                    pltpu.make_async_copy(buf1, acc.at[dstv.at[j1]],
                                          sem1).wait()

                    @pl.when(j1 + 2 < NB)
                    def _():
                        gather(j1 + 2, buf1, sem1)

                    return carry

                lax.fori_loop(0, NB // 2, body, 0)

            plsc.subcore_barrier()
            pltpu.sync_copy(acc.at[pl.ds(base, RPT)],
                            out_ref.at[pl.ds(base, RPT)])

        @pl.when(c == 0)
        def _():
            run(split_ms[0], o0, 0)

        @pl.when(c == 1)
        def _():
            run(split_ms[-1], o1, 0 if split == "feat" else nblk)

    @_deco
    def prop(src_r, dst_r, m0, m1, o0, o1, srcv, dstv, buf0, buf1, acc,
             sem0, sem1):
        body_common((m0, m1), src_r, dst_r, o0, o1, srcv, dstv, buf0,
                    buf1, acc, sem0, sem1)

    return prop


_prop_feat = _make_prop("feat")
_prop_edge = _make_prop("edge")
_prop_edge64 = _make_prop("edge", 64)


def _leaky(x):
    return jnp.where(x > 0, x, 0.1 * x)


def _nrm(ref):
    return lax.rsqrt(jnp.maximum(ref[...], 1.0))


def _stage_in_body(x_ref, w_ref, dgo_ref, o0, o1):
    m = jnp.dot(x_ref[...], w_ref[...],
                preferred_element_type=jnp.float32) * _nrm(dgo_ref)
    o0[...] = m[:, :D2]
    o1[...] = m[:, D2:]


_BM = 1000
_GRID = N // _BM


def _row_spec(d):
    return pl.BlockSpec((_BM, d), lambda i: (i, 0))


def _full_spec(shape):
    return pl.BlockSpec(shape, lambda i: (0,) * len(shape))


def _stage_in(X, W, dgo):
    return pl.pallas_call(
        _stage_in_body,
        grid=(_GRID,),
        in_specs=[_row_spec(256), _full_spec((256, 256)), _row_spec(1)],
        out_specs=[_row_spec(D2)] * 2,
        out_shape=[jax.ShapeDtypeStruct((N, D2), jnp.float32)] * 2,
    )(X, W, dgo)


def _stage_mid1_body(a0, a1, dgi, dgo, b_ref, wa, wb, o_ref):
    n_in = _nrm(dgi)
    h0 = _leaky(a0[...] * n_in + b_ref[...][:, :D2])
    h1 = _leaky(a1[...] * n_in + b_ref[...][:, D2:])
    o_ref[...] = (jnp.dot(h0, wa[...], preferred_element_type=jnp.float32)
                  + jnp.dot(h1, wb[...], preferred_element_type=jnp.float32)
                  ) * _nrm(dgo)


def _stage_mid1(a0, a1, dgi, dgo, b, Wa, Wb):
    return pl.pallas_call(
        _stage_mid1_body,
        grid=(_GRID,),
        in_specs=[_row_spec(D2), _row_spec(D2), _row_spec(1), _row_spec(1),
                  _full_spec((1, 256)), _full_spec((D2, D2)),
                  _full_spec((D2, D2))],
        out_specs=_row_spec(D2),
        out_shape=jax.ShapeDtypeStruct((N, D2), jnp.float32),
    )(a0, a1, dgi, dgo, b, Wa, Wb)


def _stage_mid2_body(p0, p1, dgi, dgo, b_ref, w3_ref, o_ref):
    h = _leaky((p0[...] + p1[...]) * _nrm(dgi) + b_ref[...])
    o_ref[...] = jnp.dot(h, w3_ref[...],
                         preferred_element_type=jnp.float32) * _nrm(dgo)


def _stage_mid2(p0, p1, dgi, dgo, b, W3):
    return pl.pallas_call(
        _stage_mid2_body,
        grid=(_GRID,),
        in_specs=[_row_spec(D2), _row_spec(D2), _row_spec(1), _row_spec(1),
                  _full_spec((1, D2)), _full_spec((D2, 64))],
        out_specs=_row_spec(64),
        out_shape=jax.ShapeDtypeStruct((N, 64), jnp.float32),
    )(p0, p1, dgi, dgo, b, W3)


def _stage_out_body(q0, q1, dgi, b_ref, sp_ref, o_ref):
    h3 = (q0[...] + q1[...]) * _nrm(dgi) + b_ref[...]
    lt = lax.dot_general(sp_ref[...], h3, (((1,), (1,)), ((), ())),
                         preferred_element_type=jnp.float32)
    row = lax.broadcasted_iota(jnp.int32, lt.shape, 0)
    lt = jnp.where(row < 500, lt, -1e30)
    mx = jnp.max(lt, axis=0, keepdims=True)
    e = jnp.exp(lt - mx)
    o_ref[...] = e / jnp.sum(e, axis=0, keepdims=True)


def _stage_out(q0, q1, dgi, b3, S_pad):
    return pl.pallas_call(
        _stage_out_body,
        out_shape=jax.ShapeDtypeStruct((512, N), jnp.float32),
    )(q0, q1, dgi, b3, S_pad)


def kernel(X, edge_index, S, W1, b1, W2, b2, W3, b3):
    ei = edge_index.astype(jnp.int32)
    src_r = ei[0].reshape(T, NC, CK)
    dst_r = ei[1].reshape(T, NC, CK)
    src_p = jnp.pad(ei[0].reshape(T, E // T),
                    ((0, 0), (0, NCP * CKP - E // T))).reshape(T, NCP, CKP)
    dst_p = jnp.pad(ei[1].reshape(T, E // T),
                    ((0, 0), (0, NCP * CKP - E // T)),
                    constant_values=N).reshape(T, NCP, CKP)

    dego_p, degi_p = _deg_kernel(src_r, dst_r)
    dgo = dego_p[:N].reshape(N, 1)
    dgi = degi_p[:N].reshape(N, 1)

    m0, m1 = _stage_in(X, W1, dgo)
    a0, a1 = _prop_feat(src_p, dst_p, m0, m1)

    m2 = _stage_mid1(a0[:N], a1[:N], dgi, dgo, b1.reshape(1, -1),
                     W2[:D2], W2[D2:])
    p0, p1 = _prop_edge(src_p, dst_p, m2, m2)

    m3 = _stage_mid2(p0[:N], p1[:N], dgi, dgo, b2.reshape(1, -1), W3)
    q0, q1 = _prop_edge64(src_p, dst_p, m3, m3)

    S_pad = jnp.pad(S, ((0, 12), (0, 0)))
    out_t = _stage_out(q0[:N], q1[:N], dgi, b3.reshape(1, -1), S_pad)
    return out_t[:500].T


# speedup vs baseline: 1.8275x; 1.8275x over previous
import functools

import jax
import jax.numpy as jnp
from jax import lax
from jax.experimental import pallas as pl
from jax.experimental.pallas import tpu as pltpu
from jax.experimental.pallas import tpu_sc as plsc

N = 10000
NPAD = 10240
E = 160000
T = 16
CK = 100
NC = (E // T) // CK
NB = 10
RPT = NPAD // T
D2 = 128

_mesh = plsc.VectorSubcoreMesh(core_axis_name="c", subcore_axis_name="s")


@functools.partial(
    pl.kernel,
    mesh=_mesh,
    out_type=[jax.ShapeDtypeStruct((NPAD,), jnp.float32)] * 2,
    scratch_types=[
        pltpu.VMEM((NC, CK), jnp.int32),
        pltpu.VMEM((CK,), jnp.float32),
        pltpu.VMEM((RPT,), jnp.float32),
        pltpu.VMEM_SHARED((NPAD,), jnp.float32),
    ],
)
def _deg_kernel(src_r, dst_r, dego, degi, idxv, onesv, zv, acc):
    c = lax.axis_index("c")
    s = lax.axis_index("s")

    def fill(ref, n, val):
        def body(i, carry):
            ref[pl.ds(i * 16, 16)] = jnp.full((16,), val, jnp.float32)
            return carry
        lax.fori_loop(0, n // 16, body, 0)
        if n % 16:
            ref[pl.ds(n - 16, 16)] = jnp.full((16,), val, jnp.float32)

    fill(onesv, CK, 1.0)
    fill(zv, RPT, 0.0)
    pltpu.sync_copy(zv, acc.at[pl.ds(s * RPT, RPT)])
    plsc.subcore_barrier()

    def run(e_r, out_ref):
        def body(g, carry):
            pltpu.sync_copy(onesv, acc.at[idxv.at[g]], add=True)
            return carry

        pltpu.sync_copy(e_r.at[s], idxv)
        lax.fori_loop(0, NC, body, 0)
        plsc.subcore_barrier()
        pltpu.sync_copy(acc.at[pl.ds(s * RPT, RPT)],
                        out_ref.at[pl.ds(s * RPT, RPT)])

    @pl.when(c == 0)
    def _():
        run(src_r, dego)

    @pl.when(c == 1)
    def _():
        run(dst_r, degi)


def _make_prop(split, d2=D2):
    nblk = (NC if split == "feat" else NC // 2) // NB

    _deco = functools.partial(
        pl.kernel,
        mesh=_mesh,
        compiler_params=pltpu.CompilerParams(use_tc_tiling_on_sc=False),
        out_type=[jax.ShapeDtypeStruct((NPAD, d2), jnp.float32)] * 2,
        scratch_types=[
            pltpu.VMEM((NB, CK), jnp.int32),
            pltpu.VMEM((NB, CK), jnp.int32),
            pltpu.VMEM((CK, d2), jnp.float32),
            pltpu.VMEM((CK, d2), jnp.float32),
            pltpu.VMEM_SHARED((NPAD, d2), jnp.float32),
            pltpu.SemaphoreType.DMA,
            pltpu.SemaphoreType.DMA,
        ],
    )

    def body_common(split_ms, src_r, dst_r, o0, o1, srcv, dstv, buf0, buf1,
                    acc, sem0, sem1):
        c = lax.axis_index("c")
        s = lax.axis_index("s")

        def zrow(i, carry):
            def zcol(j, carry2):
                buf0[i, pl.ds(j * 16, 16)] = jnp.zeros((16,), jnp.float32)
                return carry2
            lax.fori_loop(0, d2 // 16, zcol, 0)
            return carry

        lax.fori_loop(0, CK, zrow, 0)
        base = s * RPT
        for q in range(RPT // CK):
            pltpu.sync_copy(buf0, acc.at[pl.ds(base + q * CK, CK)])
        if RPT % CK:
            pltpu.sync_copy(buf0.at[pl.ds(0, RPT % CK)],
                            acc.at[pl.ds(base + (RPT // CK) * CK, RPT % CK)])
        plsc.subcore_barrier()

        def run(m_ref, out_ref, blk0):
            def gather(j, buf, sem):
                return pltpu.async_copy(m_ref.at[srcv.at[j]], buf, sem)

            def scat(j, buf, sem):
                return pltpu.async_copy(buf, acc.at[dstv.at[j]], sem,
                                        add=True)

            for blk in range(nblk):
                pltpu.sync_copy(src_r.at[s, pl.ds((blk0 + blk) * NB, NB)], srcv)
                pltpu.sync_copy(dst_r.at[s, pl.ds((blk0 + blk) * NB, NB)], dstv)
                gather(0, buf0, sem0)
                gather(1, buf1, sem1)

                def body(i, carry):
                    j0 = 2 * i
                    j1 = j0 + 1
                    pltpu.make_async_copy(m_ref.at[srcv.at[j0]], buf0,
                                          sem0).wait()
                    scat(j0, buf0, sem0)
                    pltpu.make_async_copy(m_ref.at[srcv.at[j1]], buf1,
                                          sem1).wait()
                    scat(j1, buf1, sem1)
                    pltpu.make_async_copy(buf0, acc.at[dstv.at[j0]],
                                          sem0).wait()

                    @pl.when(j0 + 2 < NB)
                    def _():
                        gather(j0 + 2, buf0, sem0)

                    pltpu.make_async_copy(buf1, acc.at[dstv.at[j1]],
                                          sem1).wait()

                    @pl.when(j1 + 2 < NB)
                    def _():
                        gather(j1 + 2, buf1, sem1)

                    return carry

                lax.fori_loop(0, NB // 2, body, 0)

            plsc.subcore_barrier()
            pltpu.sync_copy(acc.at[pl.ds(base, RPT)],
                            out_ref.at[pl.ds(base, RPT)])

        @pl.when(c == 0)
        def _():
            run(split_ms[0], o0, 0)

        @pl.when(c == 1)
        def _():
            run(split_ms[-1], o1, 0 if split == "feat" else nblk)

    @_deco
    def prop(src_r, dst_r, m0, m1, o0, o1, srcv, dstv, buf0, buf1, acc,
             sem0, sem1):
        body_common((m0, m1), src_r, dst_r, o0, o1, srcv, dstv, buf0,
                    buf1, acc, sem0, sem1)

    return prop


_prop_feat = _make_prop("feat")
_prop_edge = _make_prop("edge")
_prop_edge64 = _make_prop("edge", 64)


def _leaky(x):
    return jnp.where(x > 0, x, 0.1 * x)


def _nrm(ref):
    return lax.rsqrt(jnp.maximum(ref[...], 1.0))


def _stage_in_body(x_ref, w_ref, dgo_ref, o0, o1):
    m = jnp.dot(x_ref[...], w_ref[...],
                preferred_element_type=jnp.float32) * _nrm(dgo_ref)
    o0[...] = m[:, :D2]
    o1[...] = m[:, D2:]


_BM = 1000
_GRID = N // _BM


def _row_spec(d):
    return pl.BlockSpec((_BM, d), lambda i: (i, 0))


def _full_spec(shape):
    return pl.BlockSpec(shape, lambda i: (0,) * len(shape))


def _stage_in(X, W, dgo):
    return pl.pallas_call(
        _stage_in_body,
        grid=(_GRID,),
        in_specs=[_row_spec(256), _full_spec((256, 256)), _row_spec(1)],
        out_specs=[_row_spec(D2)] * 2,
        out_shape=[jax.ShapeDtypeStruct((N, D2), jnp.float32)] * 2,
    )(X, W, dgo)


def _stage_mid1_body(a0, a1, dgi, dgo, b_ref, wa, wb, o_ref):
    n_in = _nrm(dgi)
    h0 = _leaky(a0[...] * n_in + b_ref[...][:, :D2])
    h1 = _leaky(a1[...] * n_in + b_ref[...][:, D2:])
    o_ref[...] = (jnp.dot(h0, wa[...], preferred_element_type=jnp.float32)
                  + jnp.dot(h1, wb[...], preferred_element_type=jnp.float32)
                  ) * _nrm(dgo)


def _stage_mid1(a0, a1, dgi, dgo, b, Wa, Wb):
    return pl.pallas_call(
        _stage_mid1_body,
        grid=(_GRID,),
        in_specs=[_row_spec(D2), _row_spec(D2), _row_spec(1), _row_spec(1),
                  _full_spec((1, 256)), _full_spec((D2, D2)),
                  _full_spec((D2, D2))],
        out_specs=_row_spec(D2),
        out_shape=jax.ShapeDtypeStruct((N, D2), jnp.float32),
    )(a0, a1, dgi, dgo, b, Wa, Wb)


def _stage_mid2_body(p0, p1, dgi, dgo, b_ref, w3_ref, o_ref):
    h = _leaky((p0[...] + p1[...]) * _nrm(dgi) + b_ref[...])
    o_ref[...] = jnp.dot(h, w3_ref[...],
                         preferred_element_type=jnp.float32) * _nrm(dgo)


def _stage_mid2(p0, p1, dgi, dgo, b, W3):
    return pl.pallas_call(
        _stage_mid2_body,
        grid=(_GRID,),
        in_specs=[_row_spec(D2), _row_spec(D2), _row_spec(1), _row_spec(1),
                  _full_spec((1, D2)), _full_spec((D2, 64))],
        out_specs=_row_spec(64),
        out_shape=jax.ShapeDtypeStruct((N, 64), jnp.float32),
    )(p0, p1, dgi, dgo, b, W3)


def _stage_out_body(q0, q1, dgi, b_ref, sp_ref, o_ref):
    h3 = (q0[...] + q1[...]) * _nrm(dgi) + b_ref[...]
    lt = lax.dot_general(sp_ref[...], h3, (((1,), (1,)), ((), ())),
                         preferred_element_type=jnp.float32)
    mx = jnp.max(lt, axis=0, keepdims=True)
    e = jnp.exp(lt - mx)
    o_ref[...] = e / jnp.sum(e, axis=0, keepdims=True)


def _stage_out(q0, q1, dgi, b3, S):
    return pl.pallas_call(
        _stage_out_body,
        out_shape=jax.ShapeDtypeStruct((500, N), jnp.float32),
    )(q0, q1, dgi, b3, S)


def kernel(X, edge_index, S, W1, b1, W2, b2, W3, b3):
    ei = edge_index.astype(jnp.int32)
    src_r = ei[0].reshape(T, NC, CK)
    dst_r = ei[1].reshape(T, NC, CK)

    dego_p, degi_p = _deg_kernel(src_r, dst_r)
    dgo = dego_p[:N].reshape(N, 1)
    dgi = degi_p[:N].reshape(N, 1)

    m0, m1 = _stage_in(X, W1, dgo)
    a0, a1 = _prop_feat(src_r, dst_r, m0, m1)

    m2 = _stage_mid1(a0[:N], a1[:N], dgi, dgo, b1.reshape(1, -1),
                     W2[:D2], W2[D2:])
    p0, p1 = _prop_edge(src_r, dst_r, m2, m2)

    m3 = _stage_mid2(p0[:N], p1[:N], dgi, dgo, b2.reshape(1, -1), W3)
    q0, q1 = _prop_edge64(src_r, dst_r, m3, m3)

    out_t = _stage_out(q0[:N], q1[:N], dgi, b3.reshape(1, -1), S)
    return out_t.T


# speedup vs baseline: 1.8503x; 1.0125x over previous
import functools

import jax
import jax.numpy as jnp
from jax import lax
from jax.experimental import pallas as pl
from jax.experimental.pallas import tpu as pltpu
from jax.experimental.pallas import tpu_sc as plsc

N = 10000
NPAD = 10240
E = 160000
T = 16
CK = 100
NC = (E // T) // CK
NB = 10
RPT = NPAD // T
D2 = 128

_mesh = plsc.VectorSubcoreMesh(core_axis_name="c", subcore_axis_name="s")


@functools.partial(
    pl.kernel,
    mesh=_mesh,
    out_type=[jax.ShapeDtypeStruct((NPAD,), jnp.float32)] * 2,
    scratch_types=[
        pltpu.VMEM((NC, CK), jnp.int32),
        pltpu.VMEM((CK,), jnp.float32),
        pltpu.VMEM((RPT,), jnp.float32),
        pltpu.VMEM_SHARED((NPAD,), jnp.float32),
    ],
)
def _deg_kernel(src_r, dst_r, dego, degi, idxv, onesv, zv, acc):
    c = lax.axis_index("c")
    s = lax.axis_index("s")

    def fill(ref, n, val):
        def body(i, carry):
            ref[pl.ds(i * 16, 16)] = jnp.full((16,), val, jnp.float32)
            return carry
        lax.fori_loop(0, n // 16, body, 0)
        if n % 16:
            ref[pl.ds(n - 16, 16)] = jnp.full((16,), val, jnp.float32)

    fill(onesv, CK, 1.0)
    fill(zv, RPT, 0.0)
    pltpu.sync_copy(zv, acc.at[pl.ds(s * RPT, RPT)])
    plsc.subcore_barrier()

    def run(e_r, out_ref):
        def body(g, carry):
            pltpu.sync_copy(onesv, acc.at[idxv.at[g]], add=True)
            return carry

        pltpu.sync_copy(e_r.at[s], idxv)
        lax.fori_loop(0, NC, body, 0)
        plsc.subcore_barrier()
        pltpu.sync_copy(acc.at[pl.ds(s * RPT, RPT)],
                        out_ref.at[pl.ds(s * RPT, RPT)])

    @pl.when(c == 0)
    def _():
        run(src_r, dego)

    @pl.when(c == 1)
    def _():
        run(dst_r, degi)


def _make_prop(split, d2=D2):
    nblk = (NC if split == "feat" else NC // 2) // NB

    _deco = functools.partial(
        pl.kernel,
        mesh=_mesh,
        compiler_params=pltpu.CompilerParams(use_tc_tiling_on_sc=False),
        out_type=[jax.ShapeDtypeStruct((NPAD, d2), jnp.float32)] * 2,
        scratch_types=[
            pltpu.VMEM((NB, CK), jnp.int32),
            pltpu.VMEM((NB, CK), jnp.int32),
            pltpu.VMEM((CK, d2), jnp.float32),
            pltpu.VMEM((CK, d2), jnp.float32),
            pltpu.VMEM_SHARED((NPAD, d2), jnp.float32),
            pltpu.SemaphoreType.DMA,
            pltpu.SemaphoreType.DMA,
        ],
    )

    def body_common(split_ms, src_r, dst_r, o0, o1, srcv, dstv, buf0, buf1,
                    acc, sem0, sem1):
        c = lax.axis_index("c")
        s = lax.axis_index("s")

        def zrow(i, carry):
            def zcol(j, carry2):
                buf0[i, pl.ds(j * 16, 16)] = jnp.zeros((16,), jnp.float32)
                return carry2
            lax.fori_loop(0, d2 // 16, zcol, 0)
            return carry

        lax.fori_loop(0, CK, zrow, 0)
        base = s * RPT
        for q in range(RPT // CK):
            pltpu.sync_copy(buf0, acc.at[pl.ds(base + q * CK, CK)])
        if RPT % CK:
            pltpu.sync_copy(buf0.at[pl.ds(0, RPT % CK)],
                            acc.at[pl.ds(base + (RPT // CK) * CK, RPT % CK)])
        plsc.subcore_barrier()

        def run(m_ref, out_ref, blk0):
            def gather(j, buf, sem):
                return pltpu.async_copy(m_ref.at[srcv.at[j]], buf, sem)

            def scat(j, buf, sem):
                return pltpu.async_copy(buf, acc.at[dstv.at[j]], sem,
                                        add=True)

            for blk in range(nblk):
                pltpu.sync_copy(src_r.at[s, pl.ds((blk0 + blk) * NB, NB)], srcv)
                pltpu.sync_copy(dst_r.at[s, pl.ds((blk0 + blk) * NB, NB)], dstv)
                gather(0, buf0, sem0)
                gather(1, buf1, sem1)

                def body(i, carry):
                    j0 = 2 * i
                    j1 = j0 + 1
                    pltpu.make_async_copy(m_ref.at[srcv.at[j0]], buf0,
                                          sem0).wait()
                    scat(j0, buf0, sem0)
                    pltpu.make_async_copy(m_ref.at[srcv.at[j1]], buf1,
                                          sem1).wait()
                    scat(j1, buf1, sem1)
                    pltpu.make_async_copy(buf0, acc.at[dstv.at[j0]],
                                          sem0).wait()

                    @pl.when(j0 + 2 < NB)
                    def _():
                        gather(j0 + 2, buf0, sem0)

                    pltpu.make_async_copy(buf1, acc.at[dstv.at[j1]],
                                          sem1).wait()

                    @pl.when(j1 + 2 < NB)
                    def _():
                        gather(j1 + 2, buf1, sem1)

                    return carry

                lax.fori_loop(0, NB // 2, body, 0)

            plsc.subcore_barrier()
            pltpu.sync_copy(acc.at[pl.ds(base, RPT)],
                            out_ref.at[pl.ds(base, RPT)])

        @pl.when(c == 0)
        def _():
            run(split_ms[0], o0, 0)

        @pl.when(c == 1)
        def _():
            run(split_ms[-1], o1, 0 if split == "feat" else nblk)

    @_deco
    def prop(src_r, dst_r, m0, m1, o0, o1, srcv, dstv, buf0, buf1, acc,
             sem0, sem1):
        body_common((m0, m1), src_r, dst_r, o0, o1, srcv, dstv, buf0,
                    buf1, acc, sem0, sem1)

    return prop


_prop_feat = _make_prop("feat")
_prop_edge = _make_prop("edge")
_prop_edge64 = _make_prop("edge", 64)


def _leaky(x):
    return jnp.where(x > 0, x, 0.1 * x)


def _nrm(ref):
    return lax.rsqrt(jnp.maximum(ref[...], 1.0))


def _stage_in_body(x_ref, w_ref, dgo_ref, o0, o1):
    m = jnp.dot(x_ref[...], w_ref[...],
                preferred_element_type=jnp.float32) * _nrm(dgo_ref)
    o0[...] = m[:, :D2]
    o1[...] = m[:, D2:]


_BM = 2000
_GRID = N // _BM


def _row_spec(d):
    return pl.BlockSpec((_BM, d), lambda i: (i, 0))


def _full_spec(shape):
    return pl.BlockSpec(shape, lambda i: (0,) * len(shape))


def _stage_in(X, W, dgo):
    return pl.pallas_call(
        _stage_in_body,
        grid=(_GRID,),
        in_specs=[_row_spec(256), _full_spec((256, 256)), _row_spec(1)],
        out_specs=[_row_spec(D2)] * 2,
        out_shape=[jax.ShapeDtypeStruct((N, D2), jnp.float32)] * 2,
    )(X, W, dgo)


def _stage_mid1_body(a0, a1, dgi, dgo, b_ref, wa, wb, o_ref):
    n_in = _nrm(dgi)
    h0 = _leaky(a0[...] * n_in + b_ref[...][:, :D2])
    h1 = _leaky(a1[...] * n_in + b_ref[...][:, D2:])
    o_ref[...] = (jnp.dot(h0, wa[...], preferred_element_type=jnp.float32)
                  + jnp.dot(h1, wb[...], preferred_element_type=jnp.float32)
                  ) * _nrm(dgo)


def _stage_mid1(a0, a1, dgi, dgo, b, Wa, Wb):
    return pl.pallas_call(
        _stage_mid1_body,
        grid=(_GRID,),
        in_specs=[_row_spec(D2), _row_spec(D2), _row_spec(1), _row_spec(1),
                  _full_spec((1, 256)), _full_spec((D2, D2)),
                  _full_spec((D2, D2))],
        out_specs=_row_spec(D2),
        out_shape=jax.ShapeDtypeStruct((N, D2), jnp.float32),
    )(a0, a1, dgi, dgo, b, Wa, Wb)


def _stage_mid2_body(p0, p1, dgi, dgo, b_ref, w3_ref, o_ref):
    h = _leaky((p0[...] + p1[...]) * _nrm(dgi) + b_ref[...])
    o_ref[...] = jnp.dot(h, w3_ref[...],
                         preferred_element_type=jnp.float32) * _nrm(dgo)


def _stage_mid2(p0, p1, dgi, dgo, b, W3):
    return pl.pallas_call(
        _stage_mid2_body,
        grid=(_GRID,),
        in_specs=[_row_spec(D2), _row_spec(D2), _row_spec(1), _row_spec(1),
                  _full_spec((1, D2)), _full_spec((D2, 64))],
        out_specs=_row_spec(64),
        out_shape=jax.ShapeDtypeStruct((N, 64), jnp.float32),
    )(p0, p1, dgi, dgo, b, W3)


def _stage_out_body(q0, q1, dgi, b_ref, sp_ref, o_ref):
    h3 = (q0[...] + q1[...]) * _nrm(dgi) + b_ref[...]
    lt = lax.dot_general(sp_ref[...], h3, (((1,), (1,)), ((), ())),
                         preferred_element_type=jnp.float32)
    mx = jnp.max(lt, axis=0, keepdims=True)
    e = jnp.exp(lt - mx)
    o_ref[...] = e / jnp.sum(e, axis=0, keepdims=True)


def _stage_out(q0, q1, dgi, b3, S):
    return pl.pallas_call(
        _stage_out_body,
        out_shape=jax.ShapeDtypeStruct((500, N), jnp.float32),
    )(q0, q1, dgi, b3, S)


def kernel(X, edge_index, S, W1, b1, W2, b2, W3, b3):
    ei = edge_index.astype(jnp.int32)
    src_r = ei[0].reshape(T, NC, CK)
    dst_r = ei[1].reshape(T, NC, CK)

    dego_p, degi_p = _deg_kernel(src_r, dst_r)
    dgo = dego_p[:N].reshape(N, 1)
    dgi = degi_p[:N].reshape(N, 1)

    m0, m1 = _stage_in(X, W1, dgo)
    a0, a1 = _prop_feat(src_r, dst_r, m0, m1)

    m2 = _stage_mid1(a0[:N], a1[:N], dgi, dgo, b1.reshape(1, -1),
                     W2[:D2], W2[D2:])
    p0, p1 = _prop_edge(src_r, dst_r, m2, m2)

    m3 = _stage_mid2(p0[:N], p1[:N], dgi, dgo, b2.reshape(1, -1), W3)
    q0, q1 = _prop_edge64(src_r, dst_r, m3, m3)

    out_t = _stage_out(q0[:N], q1[:N], dgi, b3.reshape(1, -1), S)
    return out_t.T
